# Initial kernel scaffold; baseline (speedup 1.0000x reference)
import jax
import jax.numpy as jnp
from jax.experimental import pallas as pl


def kernel(input, bit_arr, codebook):
    raise NotImplementedError("write your pallas kernel here")


# speedup vs baseline: 3.1329x; 3.1329x over previous
import functools

import jax
import jax.numpy as jnp
from jax import lax
from jax.experimental import pallas as pl
from jax.experimental.pallas import tpu as pltpu
from jax.experimental.pallas import tpu_sc as plsc

_VOCAB = 1000000
_NUM_CODES = 256
_EMBED_DIM = 64
_BATCH = 4096
_FIELDS = 26

_TOTAL = _BATCH * _FIELDS
_NW = 32
_PER_W = _TOTAL // _NW
_CHUNK = 128
_NCH = _PER_W // _CHUNK
_ROW_PAD = 128


def _body(ids_hbm, bits_hbm, cb_hbm, out_hbm,
          idx_v, codes_v, rows0, rows1, csem, sem0, sem1):
    wid = lax.axis_index("s") * 2 + lax.axis_index("c")
    base = wid * _PER_W

    pltpu.sync_copy(ids_hbm.at[wid], idx_v)

    def fire_codes(j, _):
        pltpu.async_copy(bits_hbm.at[idx_v.at[j]], codes_v.at[j], csem)
        return _
    lax.fori_loop(0, _NCH, fire_codes, None)
    pltpu.make_async_copy(ids_hbm.at[wid], codes_v, csem).wait()

    def rows_start(j, buf, sem):
        pltpu.async_copy(cb_hbm.at[codes_v.at[j]], buf, sem)

    def rows_wait(j, buf, sem):
        pltpu.make_async_copy(cb_hbm.at[codes_v.at[j]], buf, sem).wait()

    def flush(j, buf):
        pltpu.sync_copy(buf, out_hbm.at[pl.ds(base + j * _CHUNK, _CHUNK)])

    rows_start(0, rows0, sem0)

    def pipe(gg, _):
        j0 = 2 * gg
        j1 = j0 + 1
        rows_start(j1, rows1, sem1)
        rows_wait(j0, rows0, sem0)
        flush(j0, rows0)
        rows_start(j0 + 2, rows0, sem0)
        rows_wait(j1, rows1, sem1)
        flush(j1, rows1)
        return _
    lax.fori_loop(0, _NCH // 2 - 1, pipe, None)

    rows_start(_NCH - 1, rows1, sem1)
    rows_wait(_NCH - 2, rows0, sem0)
    flush(_NCH - 2, rows0)
    rows_wait(_NCH - 1, rows1, sem1)
    flush(_NCH - 1, rows1)


@functools.partial(jax.jit, static_argnums=())
def kernel(input, bit_arr, codebook):
    ids = input.reshape(_NW, _NCH, _CHUNK)
    mesh = plsc.VectorSubcoreMesh(core_axis_name="c", subcore_axis_name="s")
    run = pl.kernel(
        _body,
        out_type=jax.ShapeDtypeStruct((_TOTAL, _EMBED_DIM), jnp.float32),
        mesh=mesh,
        scratch_types=[
            pltpu.VMEM((_NCH, _CHUNK), jnp.int32),
            pltpu.VMEM((_NCH, _CHUNK), jnp.int32),
            pltpu.VMEM((_CHUNK, _EMBED_DIM), jnp.float32),
            pltpu.VMEM((_CHUNK, _EMBED_DIM), jnp.float32),
            pltpu.SemaphoreType.DMA,
            pltpu.SemaphoreType.DMA,
            pltpu.SemaphoreType.DMA,
        ],
        compiler_params=pltpu.CompilerParams(use_tc_tiling_on_sc=False),
    )
    out = run(ids, bit_arr, codebook)
    return out.reshape(_BATCH, _FIELDS, _EMBED_DIM)


# speedup vs baseline: 3.9315x; 1.2549x over previous
import functools

import jax
import jax.numpy as jnp
from jax import lax
from jax.experimental import pallas as pl
from jax.experimental.pallas import tpu as pltpu
from jax.experimental.pallas import tpu_sc as plsc

_VOCAB = 1000000
_NUM_CODES = 256
_EMBED_DIM = 64
_BATCH = 4096
_FIELDS = 26

_TOTAL = _BATCH * _FIELDS
_NW = 32
_PER_W = _TOTAL // _NW
_CHUNK = 128
_NCH = _PER_W // _CHUNK

_BB = 512
_GRID = _BATCH // _BB
_LOOK = _BB * _FIELDS
_CROWS = _LOOK // _CHUNK


def _sc_codes_body(ids_hbm, bits_hbm, codes_hbm, idx_v, codes_v, csem):
    wid = lax.axis_index("s") * 2 + lax.axis_index("c")

    pltpu.sync_copy(ids_hbm.at[wid], idx_v)

    def fire(j, carry):
        pltpu.async_copy(bits_hbm.at[idx_v.at[j]], codes_v.at[j], csem)
        return carry
    lax.fori_loop(0, _NCH, fire, None)
    pltpu.make_async_copy(ids_hbm.at[wid], codes_v, csem).wait()

    pltpu.sync_copy(codes_v, codes_hbm.at[pl.ds(wid * _NCH, _NCH)])


def _tc_decode_body(codes_ref, cbh_ref, cbl_ref, out_ref, acc_ref):
    c = codes_ref[...]
    bc = jnp.broadcast_to(c, (_NUM_CODES, _CROWS, _CHUNK))
    iota = lax.broadcasted_iota(jnp.int32, (_NUM_CODES, _CROWS, _CHUNK), 0)
    oht = (bc == iota).astype(jnp.bfloat16).reshape(_NUM_CODES, _LOOK)
    dn = (((0,), (0,)), ((), ()))
    acc = lax.dot_general(oht, cbh_ref[...], dn,
                          preferred_element_type=jnp.float32)
    acc = acc + lax.dot_general(oht, cbl_ref[...], dn,
                                preferred_element_type=jnp.float32)
    acc_ref[...] = acc

    def regroup(t, carry):
        for u in range(4):
            out_ref[4 * t + u] = acc_ref[pl.ds(104 * t + 26 * u, _FIELDS), :]
        return carry
    lax.fori_loop(0, _LOOK // 104, regroup, None)


@jax.jit
def kernel(input, bit_arr, codebook):
    ids = input.reshape(_NW, _NCH, _CHUNK)

    mesh = plsc.VectorSubcoreMesh(core_axis_name="c", subcore_axis_name="s")
    sc_codes = pl.kernel(
        _sc_codes_body,
        out_type=jax.ShapeDtypeStruct((_NW * _NCH, _CHUNK), jnp.int32),
        mesh=mesh,
        scratch_types=[
            pltpu.VMEM((_NCH, _CHUNK), jnp.int32),
            pltpu.VMEM((_NCH, _CHUNK), jnp.int32),
            pltpu.SemaphoreType.DMA,
        ],
        compiler_params=pltpu.CompilerParams(use_tc_tiling_on_sc=False),
    )
    codes = sc_codes(ids, bit_arr)

    cb_hi = codebook.astype(jnp.bfloat16)
    cb_lo = (codebook - cb_hi.astype(jnp.float32)).astype(jnp.bfloat16)

    decode = pl.pallas_call(
        _tc_decode_body,
        grid=(_GRID,),
        in_specs=[
            pl.BlockSpec((1, _CROWS, _CHUNK), lambda i: (i, 0, 0)),
            pl.BlockSpec((_NUM_CODES, _EMBED_DIM), lambda i: (0, 0)),
            pl.BlockSpec((_NUM_CODES, _EMBED_DIM), lambda i: (0, 0)),
        ],
        out_specs=pl.BlockSpec((_BB, _FIELDS, _EMBED_DIM), lambda i: (i, 0, 0)),
        out_shape=jax.ShapeDtypeStruct((_BATCH, _FIELDS, _EMBED_DIM),
                                       jnp.float32),
        scratch_shapes=[pltpu.VMEM((_LOOK, _EMBED_DIM), jnp.float32)],
    )
    return decode(codes.reshape(_GRID, _CROWS, _CHUNK), cb_hi, cb_lo)


# speedup vs baseline: 10.1433x; 2.5800x over previous
import jax
import jax.numpy as jnp
from jax import lax
from jax.experimental import pallas as pl
from jax.experimental.pallas import tpu as pltpu
from jax.experimental.pallas import tpu_sc as plsc

_VOCAB = 1000000
_NUM_CODES = 256
_EMBED_DIM = 64
_BATCH = 4096
_FIELDS = 26

_TOTAL = _BATCH * _FIELDS
_NW = 32
_PER_W = _TOTAL // _NW
_CHUNK = 128
_NCH = _PER_W // _CHUNK

_BB = 512
_GRID = _BATCH // _BB


def _sc_codes_body(ids_hbm, bits_hbm, codes_hbm, idx_v, codes_v, csem):
    wid = lax.axis_index("s") * 2 + lax.axis_index("c")

    pltpu.sync_copy(ids_hbm.at[wid], idx_v)

    def fire(j, carry):
        pltpu.async_copy(bits_hbm.at[idx_v.at[j]], codes_v.at[j], csem)
        return carry
    lax.fori_loop(0, _NCH, fire, None)
    pltpu.make_async_copy(ids_hbm.at[wid], codes_v, csem).wait()

    pltpu.sync_copy(codes_v, codes_hbm.at[pl.ds(wid * _NCH, _NCH)])


def _tc_decode_body(codes_ref, cb_ref, out_ref):
    i = pl.program_id(0)
    iota = lax.broadcasted_iota(jnp.int32, (2 * _NUM_CODES, _BB), 0)
    cls = iota & (_NUM_CODES - 1)
    dn = (((0,), (0,)), ((), ()))
    for f in range(_FIELDS):
        row = codes_ref[pl.ds(f, 1), pl.ds(i * _BB, _BB)]
        bc = jnp.broadcast_to(row, (2 * _NUM_CODES, _BB))
        oht = (bc == cls).astype(jnp.bfloat16)
        acc = lax.dot_general(cb_ref[...], oht, dn,
                              preferred_element_type=jnp.float32)
        out_ref[f] = acc


@jax.jit
def kernel(input, bit_arr, codebook):
    ids = input.T.reshape(_NW, _NCH, _CHUNK)

    mesh = plsc.VectorSubcoreMesh(core_axis_name="c", subcore_axis_name="s")
    sc_codes = pl.kernel(
        _sc_codes_body,
        out_type=jax.ShapeDtypeStruct((_NW * _NCH, _CHUNK), jnp.int32),
        mesh=mesh,
        scratch_types=[
            pltpu.VMEM((_NCH, _CHUNK), jnp.int32),
            pltpu.VMEM((_NCH, _CHUNK), jnp.int32),
            pltpu.SemaphoreType.DMA,
        ],
        compiler_params=pltpu.CompilerParams(use_tc_tiling_on_sc=False),
    )
    codes = sc_codes(ids, bit_arr)

    cb_hi = codebook.astype(jnp.bfloat16)
    cb_lo = (codebook - cb_hi.astype(jnp.float32)).astype(jnp.bfloat16)
    cb_cat = jnp.concatenate([cb_hi, cb_lo], axis=0)

    decode = pl.pallas_call(
        _tc_decode_body,
        grid=(_GRID,),
        in_specs=[
            pl.BlockSpec((_FIELDS, _BATCH), lambda i: (0, 0)),
            pl.BlockSpec((2 * _NUM_CODES, _EMBED_DIM), lambda i: (0, 0)),
        ],
        out_specs=pl.BlockSpec((_FIELDS, _EMBED_DIM, _BB), lambda i: (0, 0, i)),
        out_shape=jax.ShapeDtypeStruct((_FIELDS, _EMBED_DIM, _BATCH),
                                       jnp.float32),
    )
    out_t = decode(codes.reshape(_FIELDS, _BATCH), cb_cat)
    return jnp.transpose(out_t, (2, 0, 1))


# speedup vs baseline: 10.6322x; 1.0482x over previous
import jax
import jax.numpy as jnp
from jax import lax
from jax.experimental import pallas as pl
from jax.experimental.pallas import tpu as pltpu
from jax.experimental.pallas import tpu_sc as plsc

_VOCAB = 1000000
_NUM_CODES = 256
_EMBED_DIM = 64
_BATCH = 4096
_FIELDS = 26

_TOTAL = _BATCH * _FIELDS
_NW = 32
_PER_W = _TOTAL // _NW
_CHUNK = 128
_NCH = _PER_W // _CHUNK

_BB = 512
_GRID = _BATCH // _BB


def _sc_codes_body(ids_hbm, bits_hbm, codes_hbm, idx_v, codes_v, csem):
    wid = lax.axis_index("s") * 2 + lax.axis_index("c")

    pltpu.sync_copy(ids_hbm.at[wid], idx_v)

    def fire(j, carry):
        pltpu.async_copy(bits_hbm.at[idx_v.at[j]], codes_v.at[j], csem)
        return carry
    lax.fori_loop(0, _NCH, fire, None)
    pltpu.make_async_copy(ids_hbm.at[wid], codes_v, csem).wait()

    pltpu.sync_copy(codes_v, codes_hbm.at[pl.ds(wid * _NCH, _NCH)])


_RPB = _BB // _CHUNK


def _tc_decode_body(codes_ref, cbt_ref, out_ref):
    i = pl.program_id(0)
    cls = lax.broadcasted_iota(jnp.int32, (_NUM_CODES, _BB), 0)
    dn = (((1,), (0,)), ((), ()))
    for f in range(_FIELDS):
        r0 = f * (_BATCH // _CHUNK) + _RPB * i
        row = jnp.concatenate(
            [codes_ref[pl.ds(r0 + q, 1), :] for q in range(_RPB)], axis=1)
        bc = jnp.broadcast_to(row, (_NUM_CODES, _BB))
        oht = (bc == cls).astype(jnp.bfloat16)
        acc = lax.dot_general(cbt_ref[:, :_NUM_CODES], oht, dn,
                              preferred_element_type=jnp.float32)
        acc = acc + lax.dot_general(cbt_ref[:, _NUM_CODES:], oht, dn,
                                    preferred_element_type=jnp.float32)
        out_ref[f] = acc


@jax.jit
def kernel(input, bit_arr, codebook):
    ids = input.T.reshape(_NW, _NCH, _CHUNK)

    mesh = plsc.VectorSubcoreMesh(core_axis_name="c", subcore_axis_name="s")
    sc_codes = pl.kernel(
        _sc_codes_body,
        out_type=jax.ShapeDtypeStruct((_NW * _NCH, _CHUNK), jnp.int32),
        mesh=mesh,
        scratch_types=[
            pltpu.VMEM((_NCH, _CHUNK), jnp.int32),
            pltpu.VMEM((_NCH, _CHUNK), jnp.int32),
            pltpu.SemaphoreType.DMA,
        ],
        compiler_params=pltpu.CompilerParams(use_tc_tiling_on_sc=False),
    )
    codes = sc_codes(ids, bit_arr)

    cb_hi = codebook.astype(jnp.bfloat16)
    cb_lo = (codebook - cb_hi.astype(jnp.float32)).astype(jnp.bfloat16)
    cbt_cat = jnp.concatenate([cb_hi.T, cb_lo.T], axis=1)

    decode = pl.pallas_call(
        _tc_decode_body,
        grid=(_GRID,),
        in_specs=[
            pl.BlockSpec((_NW * _NCH, _CHUNK), lambda i: (0, 0)),
            pl.BlockSpec((_EMBED_DIM, 2 * _NUM_CODES), lambda i: (0, 0)),
        ],
        out_specs=pl.BlockSpec((_FIELDS, _EMBED_DIM, _BB), lambda i: (0, 0, i)),
        out_shape=jax.ShapeDtypeStruct((_FIELDS, _EMBED_DIM, _BATCH),
                                       jnp.float32),
    )
    out_t = decode(codes, cbt_cat)
    return jnp.transpose(out_t, (2, 0, 1))
